# baseline (speedup 1.0000x reference)
import jax
import jax.numpy as jnp
import numpy as np
from jax.experimental import pallas as pl
from jax.experimental.pallas import tpu as pltpu


B, L, D_IN, D_EMB = 1024, 64, 512, 1024
K = 18
BB = 8
FL = BB * L
RK = BB * K
NSTEP1 = B // BB
ROWS = B * K
R2 = 512
NSTEP2 = ROWS // R2
NEG = float("-inf")
HIGHEST = jax.lax.Precision.HIGHEST
IMIN = -2147483648


def _consts():
    i = np.arange(FL)
    same = (i[:, None] // L) == (i[None, :] // L)
    tie = same & (i[None, :] < i[:, None])
    le = same & (i[:, None] <= i[None, :])
    diag = i[:, None] == i[None, :]
    q = np.arange(RK)
    qb = (q[:, None] // K) == (i[None, :] // L)
    qs = np.tile((q % K)[:, None], (1, FL))
    f32 = lambda a: jnp.asarray(a, jnp.float32)
    return (f32(same), f32(tie), f32(le), f32(diag), f32(qb), f32(qs),
            jnp.asarray(i[:, None], jnp.int32))


def _k1(feat_ref, tr_ref, ic_ref, same_ref, tie_ref, le_ref,
        diag_ref, qb_ref, qs_ref, wg1_ref, bg1_ref, wg2_ref, wm1_ref,
        bm1_ref, z1_ref, sel_ref, zsum_ref, zsq_ref):
    f2 = feat_ref[...].reshape(FL, D_IN)
    h = jnp.maximum(jax.lax.dot_general(
        f2, wg1_ref[...], (((1,), (1,)), ((), ())),
        preferred_element_type=jnp.float32) + bg1_ref[...], 0.0)
    sameb = same_ref[...] != 0.0
    diagb = diag_ref[...] != 0.0
    g2c = jnp.sum(jnp.where(diagb, wg2_ref[...], 0.0), axis=1,
                  keepdims=True)
    g2m = jnp.where(
        jax.lax.broadcasted_iota(jnp.int32, (FL, 128), 1) == 0, g2c, 0.0)
    wcol = jnp.dot(h, g2m,
                   preferred_element_type=jnp.float32)[:, 0:1]
    tj = tr_ref[0]
    tmax = jnp.max(jnp.where(sameb, tj, IMIN), axis=1, keepdims=True)
    lanej = jax.lax.broadcasted_iota(jnp.int32, (FL, FL), 1)
    firstmax = jnp.min(jnp.where(sameb & (tj == tmax), lanej, FL),
                       axis=1, keepdims=True)
    tc = jnp.max(jnp.where(diagb, tj, IMIN), axis=1, keepdims=True)
    ic = ic_ref[...]
    kill = (ic == firstmax) | ((ic & (L - 1)) == 0) | (tc == 0)
    wcol = jnp.where(kill, NEG, wcol)

    wrow = jnp.max(jnp.where(diagb, wcol, NEG), axis=0, keepdims=True)

    beats = (jnp.where(wrow > wcol, same_ref[...], 0.0)
             + jnp.where(wrow == wcol, tie_ref[...], 0.0))
    rank = jnp.sum(beats, axis=1, keepdims=True)
    mcol = rank < float(K)
    pos = jnp.sum(jnp.where(mcol, le_ref[...], 0.0), axis=0,
                  keepdims=True) - 1.0
    mrow = jnp.sum(jnp.where(mcol & diagb, 1.0, 0.0), axis=0,
                   keepdims=True)

    p = jnp.where((pos == qs_ref[...]) & (mrow != 0.0), qb_ref[...], 0.0)
    sel = jnp.dot(p, f2, precision=HIGHEST,
                  preferred_element_type=jnp.float32)

    nrm = jnp.sqrt(jnp.sum(sel * sel, axis=1, keepdims=True)) + 1e-8
    seln = sel / nrm
    sel_ref[...] = seln

    z1 = jax.lax.dot_general(
        seln, wm1_ref[...], (((1,), (1,)), ((), ())),
        preferred_element_type=jnp.float32) + bm1_ref[...]
    z1_ref[...] = z1
    zsum_ref[...] = jnp.sum(z1, axis=0, keepdims=True)[None]
    zsq_ref[...] = jnp.sum(z1 * z1, axis=0, keepdims=True)[None]


def _k2(z1_ref, sel_ref, zsum_ref, zsq_ref, wlin_ref, blin_ref, wm2_ref,
        bm2_ref, g_ref, bt_ref, out_ref):
    n = float(ROWS)
    mu = jnp.sum(zsum_ref[...], axis=0) / n
    var = jnp.sum(zsq_ref[...], axis=0) / n - mu * mu
    rstd = jax.lax.rsqrt(var + 1e-5)
    zn = (z1_ref[...] - mu) * (rstd * g_ref[...]) + bt_ref[...]
    a = jnp.maximum(zn, 0.0)
    tdot = lambda x, w: jax.lax.dot_general(
        x, w, (((1,), (1,)), ((), ())),
        preferred_element_type=jnp.float32)
    mlp = tdot(a, wm2_ref[...]) + bm2_ref[...]
    cap = tdot(sel_ref[...].astype(jnp.bfloat16), wlin_ref[...])
    out_ref[...] = mlp + cap + blin_ref[...]


def _stage1(features, text, W_g1, b_g1, W_g2, W_m1, b_m1):
    trow = text.reshape(NSTEP1, 1, FL)
    row = lambda v: v.reshape(1, -1)
    same, tie, le, diag, qb, qs, icol = _consts()
    cst = lambda shape: pl.BlockSpec(shape, lambda i: (0,) * len(shape))

    z1, sel, zsum, zsq = pl.pallas_call(
        _k1,
        grid=(NSTEP1,),
        in_specs=[
            pl.BlockSpec((BB, L, D_IN), lambda i: (i, 0, 0)),
            pl.BlockSpec((1, 1, FL), lambda i: (i, 0, 0)),
            cst((FL, 1)),
            cst((FL, FL)),
            cst((FL, FL)),
            cst((FL, FL)),
            cst((FL, FL)),
            cst((RK, FL)),
            cst((RK, FL)),
            cst((D_IN, D_IN)),
            cst((1, D_IN)),
            cst((1, D_IN)),
            cst((D_IN, D_IN)),
            cst((1, D_IN)),
        ],
        out_specs=[
            pl.BlockSpec((RK, D_IN), lambda i: (i, 0)),
            pl.BlockSpec((RK, D_IN), lambda i: (i, 0)),
            pl.BlockSpec((1, 1, D_IN), lambda i: (i, 0, 0)),
            pl.BlockSpec((1, 1, D_IN), lambda i: (i, 0, 0)),
        ],
        out_shape=[
            jax.ShapeDtypeStruct((ROWS, D_IN), jnp.float32),
            jax.ShapeDtypeStruct((ROWS, D_IN), jnp.float32),
            jax.ShapeDtypeStruct((NSTEP1, 1, D_IN), jnp.float32),
            jax.ShapeDtypeStruct((NSTEP1, 1, D_IN), jnp.float32),
        ],
        compiler_params=pltpu.CompilerParams(
            dimension_semantics=("parallel",)),
    )(features, trow, icol, same, tie, le, diag, qb, qs,
      W_g1, row(b_g1), W_g2, W_m1, row(b_m1))
    return z1, sel, zsum, zsq


def kernel(features, text, atten, W_g1, b_g1, W_g2, b_g2, W_lin, b_lin,
           W_m1, b_m1, bn_gamma, bn_beta, W_m2, b_m2):
    del atten, b_g2
    z1, sel, zsum, zsq = _stage1(features, text, W_g1, b_g1, W_g2,
                                 W_m1, b_m1)
    row = lambda v: v.reshape(1, -1)
    cst = lambda shape: pl.BlockSpec(shape, lambda i: (0,) * len(shape))

    out = pl.pallas_call(
        _k2,
        grid=(NSTEP2,),
        in_specs=[
            pl.BlockSpec((R2, D_IN), lambda i: (i, 0)),
            pl.BlockSpec((R2, D_IN), lambda i: (i, 0)),
            cst((NSTEP1, 1, D_IN)),
            cst((NSTEP1, 1, D_IN)),
            cst((D_EMB, D_IN)),
            cst((1, D_EMB)),
            cst((D_EMB, D_IN)),
            cst((1, D_EMB)),
            cst((1, D_IN)),
            cst((1, D_IN)),
        ],
        out_specs=pl.BlockSpec((R2, D_EMB), lambda i: (i, 0)),
        out_shape=jax.ShapeDtypeStruct((ROWS, D_EMB), jnp.float32),
        compiler_params=pltpu.CompilerParams(
            dimension_semantics=("parallel",)),
    )(z1, sel, zsum, zsq, W_lin.astype(jnp.bfloat16), row(b_lin),
      W_m2, row(b_m2), row(bn_gamma), row(bn_beta))

    return out.reshape(B, K, D_EMB)


# speedup vs baseline: 1.3100x; 1.3100x over previous
import jax
import jax.numpy as jnp
import numpy as np
from jax.experimental import pallas as pl
from jax.experimental.pallas import tpu as pltpu


B, L, D_IN, D_EMB = 1024, 64, 512, 1024
K = 18
BB = 8
FL = BB * L
RK = BB * K
NSTEP1 = B // BB
ROWS = B * K
R2 = 512
NSTEP2 = ROWS // R2
NEG = float("-inf")
HIGHEST = jax.lax.Precision.HIGHEST
IMIN = -2147483648


def _consts():
    i = np.arange(FL)
    same = (i[:, None] // L) == (i[None, :] // L)
    tie = same & (i[None, :] < i[:, None])
    le = same & (i[:, None] <= i[None, :])
    diag = i[:, None] == i[None, :]
    q = np.arange(RK)
    qb = (q[:, None] % BB) == (i[None, :] // L)
    qs = np.tile((q[:, None] // BB), (1, FL))
    f32 = lambda a: jnp.asarray(a, jnp.float32)
    return (f32(same), f32(tie), f32(le), f32(diag), f32(qb), f32(qs),
            jnp.asarray(i[:, None], jnp.int32))


def _k1(feat_ref, tr_ref, ic_ref, same_ref, tie_ref, le_ref,
        diag_ref, qb_ref, qs_ref, wg1_ref, bg1_ref, wg2_ref, wm1_ref,
        bm1_ref, z1_ref, sel_ref, zsum_ref, zsq_ref):
    f2 = feat_ref[...].reshape(FL, D_IN)
    h = jnp.maximum(jax.lax.dot_general(
        f2, wg1_ref[...], (((1,), (1,)), ((), ())),
        preferred_element_type=jnp.float32) + bg1_ref[...], 0.0)
    sameb = same_ref[...] != 0.0
    diagb = diag_ref[...] != 0.0
    g2c = jnp.sum(jnp.where(diagb, wg2_ref[...], 0.0), axis=1,
                  keepdims=True)
    g2m = jnp.where(
        jax.lax.broadcasted_iota(jnp.int32, (FL, 128), 1) == 0, g2c, 0.0)
    wcol = jnp.dot(h, g2m,
                   preferred_element_type=jnp.float32)[:, 0:1]
    tj = tr_ref[0]
    tmax = jnp.max(jnp.where(sameb, tj, IMIN), axis=1, keepdims=True)
    lanej = jax.lax.broadcasted_iota(jnp.int32, (FL, FL), 1)
    firstmax = jnp.min(jnp.where(sameb & (tj == tmax), lanej, FL),
                       axis=1, keepdims=True)
    tc = jnp.max(jnp.where(diagb, tj, IMIN), axis=1, keepdims=True)
    ic = ic_ref[...]
    kill = (ic == firstmax) | ((ic & (L - 1)) == 0) | (tc == 0)
    wcol = jnp.where(kill, NEG, wcol)

    wrow = jnp.max(jnp.where(diagb, wcol, NEG), axis=0, keepdims=True)

    beats = (jnp.where(wrow > wcol, same_ref[...], 0.0)
             + jnp.where(wrow == wcol, tie_ref[...], 0.0))
    rank = jnp.sum(beats, axis=1, keepdims=True)
    mcol = rank < float(K)
    pos = jnp.sum(jnp.where(mcol, le_ref[...], 0.0), axis=0,
                  keepdims=True) - 1.0
    mrow = jnp.sum(jnp.where(mcol & diagb, 1.0, 0.0), axis=0,
                   keepdims=True)

    p = jnp.where((pos == qs_ref[...]) & (mrow != 0.0), qb_ref[...], 0.0)
    sel = jnp.dot(p, f2, precision=HIGHEST,
                  preferred_element_type=jnp.float32)

    nrm = jnp.sqrt(jnp.sum(sel * sel, axis=1, keepdims=True)) + 1e-8
    seln = sel / nrm
    sel_ref[...] = seln.reshape(K, BB, D_IN)

    z1 = jax.lax.dot_general(
        seln, wm1_ref[...], (((1,), (1,)), ((), ())),
        preferred_element_type=jnp.float32) + bm1_ref[...]
    z1_ref[...] = z1.reshape(K, BB, D_IN)
    zsum_ref[...] = jnp.sum(z1, axis=0, keepdims=True)[None]
    zsq_ref[...] = jnp.sum(z1 * z1, axis=0, keepdims=True)[None]


def _k2(z1_ref, sel_ref, zsum_ref, zsq_ref, wlin_ref, blin_ref, wm2_ref,
        bm2_ref, g_ref, bt_ref, out_ref):
    n = float(ROWS)
    mu = jnp.sum(zsum_ref[...], axis=0) / n
    var = jnp.sum(zsq_ref[...], axis=0) / n - mu * mu
    rstd = jax.lax.rsqrt(var + 1e-5)
    zn = (z1_ref[...] - mu) * (rstd * g_ref[...]) + bt_ref[...]
    a = jnp.maximum(zn, 0.0)
    tdot = lambda x, w: jax.lax.dot_general(
        x, w, (((1,), (1,)), ((), ())),
        preferred_element_type=jnp.float32)
    mlp = tdot(a, wm2_ref[...]) + bm2_ref[...]
    cap = tdot(sel_ref[...].astype(jnp.bfloat16), wlin_ref[...])
    out_ref[...] = mlp + cap + blin_ref[...]


def _stage1(features, text, W_g1, b_g1, W_g2, W_m1, b_m1):
    trow = text.reshape(NSTEP1, 1, FL)
    row = lambda v: v.reshape(1, -1)
    same, tie, le, diag, qb, qs, icol = _consts()
    cst = lambda shape: pl.BlockSpec(shape, lambda i: (0,) * len(shape))

    z1, sel, zsum, zsq = pl.pallas_call(
        _k1,
        grid=(NSTEP1,),
        in_specs=[
            pl.BlockSpec((BB, L, D_IN), lambda i: (i, 0, 0)),
            pl.BlockSpec((1, 1, FL), lambda i: (i, 0, 0)),
            cst((FL, 1)),
            cst((FL, FL)),
            cst((FL, FL)),
            cst((FL, FL)),
            cst((FL, FL)),
            cst((RK, FL)),
            cst((RK, FL)),
            cst((D_IN, D_IN)),
            cst((1, D_IN)),
            cst((1, D_IN)),
            cst((D_IN, D_IN)),
            cst((1, D_IN)),
        ],
        out_specs=[
            pl.BlockSpec((K, BB, D_IN), lambda i: (0, i, 0)),
            pl.BlockSpec((K, BB, D_IN), lambda i: (0, i, 0)),
            pl.BlockSpec((1, 1, D_IN), lambda i: (i, 0, 0)),
            pl.BlockSpec((1, 1, D_IN), lambda i: (i, 0, 0)),
        ],
        out_shape=[
            jax.ShapeDtypeStruct((K, B, D_IN), jnp.float32),
            jax.ShapeDtypeStruct((K, B, D_IN), jnp.float32),
            jax.ShapeDtypeStruct((NSTEP1, 1, D_IN), jnp.float32),
            jax.ShapeDtypeStruct((NSTEP1, 1, D_IN), jnp.float32),
        ],
        compiler_params=pltpu.CompilerParams(
            dimension_semantics=("parallel",)),
    )(features, trow, icol, same, tie, le, diag, qb, qs,
      W_g1, row(b_g1), W_g2, W_m1, row(b_m1))
    return z1, sel, zsum, zsq


def kernel(features, text, atten, W_g1, b_g1, W_g2, b_g2, W_lin, b_lin,
           W_m1, b_m1, bn_gamma, bn_beta, W_m2, b_m2):
    del atten, b_g2
    z1, sel, zsum, zsq = _stage1(features, text, W_g1, b_g1, W_g2,
                                 W_m1, b_m1)
    z1 = z1.reshape(ROWS, D_IN)
    sel = sel.reshape(ROWS, D_IN)
    row = lambda v: v.reshape(1, -1)
    cst = lambda shape: pl.BlockSpec(shape, lambda i: (0,) * len(shape))

    out = pl.pallas_call(
        _k2,
        grid=(NSTEP2,),
        in_specs=[
            pl.BlockSpec((R2, D_IN), lambda i: (i, 0)),
            pl.BlockSpec((R2, D_IN), lambda i: (i, 0)),
            cst((NSTEP1, 1, D_IN)),
            cst((NSTEP1, 1, D_IN)),
            cst((D_EMB, D_IN)),
            cst((1, D_EMB)),
            cst((D_EMB, D_IN)),
            cst((1, D_EMB)),
            cst((1, D_IN)),
            cst((1, D_IN)),
        ],
        out_specs=pl.BlockSpec((R2, D_EMB), lambda i: (i, 0)),
        out_shape=jax.ShapeDtypeStruct((ROWS, D_EMB), jnp.float32),
        compiler_params=pltpu.CompilerParams(
            dimension_semantics=("parallel",)),
    )(z1, sel, zsum, zsq, W_lin.astype(jnp.bfloat16), row(b_lin),
      W_m2, row(b_m2), row(bn_gamma), row(bn_beta))

    return out.reshape(K, B, D_EMB).transpose(1, 0, 2)


# speedup vs baseline: 1.3828x; 1.0556x over previous
import jax
import jax.numpy as jnp
import numpy as np
from jax.experimental import pallas as pl
from jax.experimental.pallas import tpu as pltpu


B, L, D_IN, D_EMB = 1024, 64, 512, 1024
K = 18
BB = 8
FL = BB * L
RK = BB * K
NSTEP1 = B // BB
ROWS = B * K
R2 = 512
NSTEP2 = ROWS // R2
NEG = float("-inf")
HIGHEST = jax.lax.Precision.HIGHEST
IMIN = -2147483648


def _consts():
    i = np.arange(FL)
    same = (i[:, None] // L) == (i[None, :] // L)
    tie = same & (i[None, :] < i[:, None])
    le = same & (i[:, None] <= i[None, :])
    diag = i[:, None] == i[None, :]
    q = np.arange(RK)
    qb = (q[:, None] % BB) == (i[None, :] // L)
    qs = np.tile((q[:, None] // BB), (1, FL))
    dg = np.arange(D_IN)
    diag_d = dg[:, None] == dg[None, :]
    f32 = lambda a: jnp.asarray(a, jnp.float32)
    return (f32(same), f32(tie), f32(le), f32(diag), f32(diag_d), f32(qb),
            f32(qs), jnp.asarray(i[:, None], jnp.int32))


def _k1(feat_ref, tr_ref, ic_ref, same_ref, tie_ref, le_ref,
        diag_ref, dgd_ref, qb_ref, qs_ref, wg1_ref, bg1_ref, wg2_ref,
        wm1_ref, bm1_ref, z1_ref, sel_ref, zsum_ref, zsq_ref):
    f2 = feat_ref[...].reshape(FL, D_IN)
    h = jnp.maximum(jax.lax.dot_general(
        f2, wg1_ref[...], (((1,), (1,)), ((), ())),
        preferred_element_type=jnp.float32) + bg1_ref[...], 0.0)
    sameb = same_ref[...] != 0.0
    diagb = diag_ref[...] != 0.0
    g2c = jnp.sum(jnp.where(dgd_ref[...] != 0.0, wg2_ref[...], 0.0),
                  axis=1, keepdims=True)
    g2m = jnp.where(
        jax.lax.broadcasted_iota(jnp.int32, (D_IN, 128), 1) == 0, g2c,
        0.0)
    wcol = jnp.dot(h, g2m,
                   preferred_element_type=jnp.float32)[:, 0:1]
    tj = tr_ref[0]
    ic = ic_ref[...]
    lanej = jax.lax.broadcasted_iota(jnp.int32, (FL, FL), 1)
    keyj = tj * L + (L - 1) - (lanej & (L - 1))
    kmax = jnp.max(jnp.where(sameb, keyj, IMIN), axis=1, keepdims=True)
    tc = jnp.max(jnp.where(diagb, tj, IMIN), axis=1, keepdims=True)
    keyc = tc * L + (L - 1) - (ic & (L - 1))
    kill = (keyc == kmax) | ((ic & (L - 1)) == 0) | (tc == 0)
    wcol = jnp.where(kill, NEG, wcol)

    wrow = jnp.max(jnp.where(diagb, wcol, NEG), axis=0, keepdims=True)

    beats = (jnp.where(wrow > wcol, same_ref[...], 0.0)
             + jnp.where(wrow == wcol, tie_ref[...], 0.0))
    rank = jnp.sum(beats, axis=1, keepdims=True)
    mcol = rank < float(K)
    cnt = jnp.sum(jnp.where(mcol, le_ref[...], 0.0), axis=0,
                  keepdims=True)
    pos = cnt - 1.0
    prev = jnp.where(
        (jax.lax.broadcasted_iota(jnp.int32, (1, FL), 1) & (L - 1)) == 0,
        0.0, jnp.roll(cnt, 1, axis=1))
    mrow = cnt - prev

    p = jnp.where((pos == qs_ref[...]) & (mrow != 0.0), qb_ref[...], 0.0)
    sel = jnp.dot(p, f2, precision=HIGHEST,
                  preferred_element_type=jnp.float32)

    nrm = jnp.sqrt(jnp.sum(sel * sel, axis=1, keepdims=True)) + 1e-8
    seln = sel / nrm
    sel_ref[...] = seln.reshape(K, BB, D_IN)

    z1 = jax.lax.dot_general(
        seln, wm1_ref[...], (((1,), (1,)), ((), ())),
        preferred_element_type=jnp.float32) + bm1_ref[...]
    z1_ref[...] = z1.reshape(K, BB, D_IN)
    zsum_ref[...] = jnp.sum(z1, axis=0, keepdims=True)[None]
    zsq_ref[...] = jnp.sum(z1 * z1, axis=0, keepdims=True)[None]


def _k2(z1_ref, sel_ref, zsum_ref, zsq_ref, wlin_ref, blin_ref, wm2_ref,
        bm2_ref, g_ref, bt_ref, out_ref):
    n = float(ROWS)
    mu = jnp.sum(zsum_ref[...], axis=0) / n
    var = jnp.sum(zsq_ref[...], axis=0) / n - mu * mu
    rstd = jax.lax.rsqrt(var + 1e-5)
    zn = (z1_ref[...] - mu) * (rstd * g_ref[...]) + bt_ref[...]
    a = jnp.maximum(zn, 0.0)
    tdot = lambda x, w: jax.lax.dot_general(
        x, w, (((1,), (1,)), ((), ())),
        preferred_element_type=jnp.float32)
    mlp = tdot(a, wm2_ref[...]) + bm2_ref[...]
    cap = tdot(sel_ref[...].astype(jnp.bfloat16), wlin_ref[...])
    out_ref[...] = mlp + cap + blin_ref[...]


def _stage1(features, text, W_g1, b_g1, W_g2, W_m1, b_m1):
    trow = text.reshape(NSTEP1, 1, FL)
    row = lambda v: v.reshape(1, -1)
    same, tie, le, diag, dgd, qb, qs, icol = _consts()
    cst = lambda shape: pl.BlockSpec(shape, lambda i: (0,) * len(shape))

    z1, sel, zsum, zsq = pl.pallas_call(
        _k1,
        grid=(NSTEP1,),
        in_specs=[
            pl.BlockSpec((BB, L, D_IN), lambda i: (i, 0, 0)),
            pl.BlockSpec((1, 1, FL), lambda i: (i, 0, 0)),
            cst((FL, 1)),
            cst((FL, FL)),
            cst((FL, FL)),
            cst((FL, FL)),
            cst((FL, FL)),
            cst((D_IN, D_IN)),
            cst((RK, FL)),
            cst((RK, FL)),
            cst((D_IN, D_IN)),
            cst((1, D_IN)),
            cst((1, D_IN)),
            cst((D_IN, D_IN)),
            cst((1, D_IN)),
        ],
        out_specs=[
            pl.BlockSpec((K, BB, D_IN), lambda i: (0, i, 0)),
            pl.BlockSpec((K, BB, D_IN), lambda i: (0, i, 0)),
            pl.BlockSpec((1, 1, D_IN), lambda i: (i, 0, 0)),
            pl.BlockSpec((1, 1, D_IN), lambda i: (i, 0, 0)),
        ],
        out_shape=[
            jax.ShapeDtypeStruct((K, B, D_IN), jnp.float32),
            jax.ShapeDtypeStruct((K, B, D_IN), jnp.float32),
            jax.ShapeDtypeStruct((NSTEP1, 1, D_IN), jnp.float32),
            jax.ShapeDtypeStruct((NSTEP1, 1, D_IN), jnp.float32),
        ],
        compiler_params=pltpu.CompilerParams(
            dimension_semantics=("parallel",)),
    )(features, trow, icol, same, tie, le, diag, dgd, qb, qs,
      W_g1, row(b_g1), W_g2, W_m1, row(b_m1))
    return z1, sel, zsum, zsq


def kernel(features, text, atten, W_g1, b_g1, W_g2, b_g2, W_lin, b_lin,
           W_m1, b_m1, bn_gamma, bn_beta, W_m2, b_m2):
    del atten, b_g2
    z1, sel, zsum, zsq = _stage1(features, text, W_g1, b_g1, W_g2,
                                 W_m1, b_m1)
    z1 = z1.reshape(ROWS, D_IN)
    sel = sel.reshape(ROWS, D_IN)
    row = lambda v: v.reshape(1, -1)
    cst = lambda shape: pl.BlockSpec(shape, lambda i: (0,) * len(shape))

    out = pl.pallas_call(
        _k2,
        grid=(NSTEP2,),
        in_specs=[
            pl.BlockSpec((R2, D_IN), lambda i: (i, 0)),
            pl.BlockSpec((R2, D_IN), lambda i: (i, 0)),
            cst((NSTEP1, 1, D_IN)),
            cst((NSTEP1, 1, D_IN)),
            cst((D_EMB, D_IN)),
            cst((1, D_EMB)),
            cst((D_EMB, D_IN)),
            cst((1, D_EMB)),
            cst((1, D_IN)),
            cst((1, D_IN)),
        ],
        out_specs=pl.BlockSpec((R2, D_EMB), lambda i: (i, 0)),
        out_shape=jax.ShapeDtypeStruct((ROWS, D_EMB), jnp.float32),
        compiler_params=pltpu.CompilerParams(
            dimension_semantics=("parallel",)),
    )(z1, sel, zsum, zsq, W_lin.astype(jnp.bfloat16), row(b_lin),
      W_m2, row(b_m2), row(bn_gamma), row(bn_beta))

    return out.reshape(K, B, D_EMB).transpose(1, 0, 2)


# speedup vs baseline: 1.6483x; 1.1920x over previous
import jax
import jax.numpy as jnp
import numpy as np
from jax.experimental import pallas as pl
from jax.experimental.pallas import tpu as pltpu


B, L, D_IN, D_EMB = 1024, 64, 512, 1024
K = 18
BB = 8
FL = BB * L
RK = BB * K
NSTEP1 = B // BB
ROWS = B * K
R2 = 512
NSTEP2 = ROWS // R2
NEG = float("-inf")
HIGHEST = jax.lax.Precision.HIGHEST
IMIN = -2147483648


def _consts():
    i = np.arange(FL)
    same = (i[:, None] // L) == (i[None, :] // L)
    tie = same & (i[None, :] < i[:, None])
    le = same & (i[:, None] <= i[None, :])
    diag = i[:, None] == i[None, :]
    q = np.arange(RK)
    qb = (q[:, None] % BB) == (i[None, :] // L)
    qs = np.tile((q[:, None] // BB), (1, FL))
    dg = np.arange(D_IN)
    diag_d = dg[:, None] == dg[None, :]
    f32 = lambda a: jnp.asarray(a, jnp.float32)
    return (f32(same), f32(tie), f32(le), f32(diag), f32(diag_d), f32(qb),
            f32(qs), jnp.asarray(i[:, None], jnp.int32))


def _k1(feat_ref, tr_ref, ic_ref, same_ref, tie_ref, le_ref,
        diag_ref, dgd_ref, qb_ref, qs_ref, wg1_ref, bg1_ref, wg2_ref,
        wm1_ref, bm1_ref, z1_ref, sel_ref, zsum_ref, zsq_ref):
    f2 = feat_ref[...].reshape(FL, D_IN)
    h = jnp.maximum(jax.lax.dot_general(
        f2, wg1_ref[...], (((1,), (1,)), ((), ())),
        preferred_element_type=jnp.float32) + bg1_ref[...], 0.0)
    sameb = same_ref[...] != 0.0
    diagb = diag_ref[...] != 0.0
    g2c = jnp.sum(jnp.where(dgd_ref[...] != 0.0, wg2_ref[...], 0.0),
                  axis=1, keepdims=True)
    g2m = jnp.where(
        jax.lax.broadcasted_iota(jnp.int32, (D_IN, 128), 1) == 0, g2c,
        0.0)
    wcol = jnp.dot(h, g2m,
                   preferred_element_type=jnp.float32)[:, 0:1]
    tj = tr_ref[0]
    ic = ic_ref[...]
    lanej = jax.lax.broadcasted_iota(jnp.int32, (FL, FL), 1)
    keyj = tj * L + (L - 1) - (lanej & (L - 1))
    kmax = jnp.max(jnp.where(sameb, keyj, IMIN), axis=1, keepdims=True)
    tc = jnp.max(jnp.where(diagb, tj, IMIN), axis=1, keepdims=True)
    keyc = tc * L + (L - 1) - (ic & (L - 1))
    kill = (keyc == kmax) | ((ic & (L - 1)) == 0) | (tc == 0)
    wcol = jnp.where(kill, NEG, wcol)

    wrow = jnp.max(jnp.where(diagb, wcol, NEG), axis=0, keepdims=True)

    beats = (jnp.where(wrow > wcol, same_ref[...], 0.0)
             + jnp.where(wrow == wcol, tie_ref[...], 0.0))
    rank = jnp.sum(beats, axis=1, keepdims=True)
    mcol = rank < float(K)
    cnt = jnp.sum(jnp.where(mcol, le_ref[...], 0.0), axis=0,
                  keepdims=True)
    pos = cnt - 1.0
    prev = jnp.where(
        (jax.lax.broadcasted_iota(jnp.int32, (1, FL), 1) & (L - 1)) == 0,
        0.0, jnp.roll(cnt, 1, axis=1))
    mrow = cnt - prev

    p = jnp.where((pos == qs_ref[...]) & (mrow != 0.0), qb_ref[...], 0.0)
    sel = jnp.dot(p, f2,
                  preferred_element_type=jnp.float32)

    nrm = jnp.sqrt(jnp.sum(sel * sel, axis=1, keepdims=True)) + 1e-8
    seln = sel / nrm
    sel_ref[...] = seln.reshape(K, BB, D_IN)

    z1 = jax.lax.dot_general(
        seln, wm1_ref[...], (((1,), (1,)), ((), ())),
        preferred_element_type=jnp.float32) + bm1_ref[...]
    z1_ref[...] = z1.reshape(K, BB, D_IN)
    zsum_ref[...] = jnp.sum(z1, axis=0, keepdims=True)[None]
    zsq_ref[...] = jnp.sum(z1 * z1, axis=0, keepdims=True)[None]


def _k2(z1_ref, sel_ref, zsum_ref, zsq_ref, wlin_ref, blin_ref, wm2_ref,
        bm2_ref, g_ref, bt_ref, out_ref):
    n = float(ROWS)
    mu = jnp.sum(zsum_ref[...], axis=0) / n
    var = jnp.sum(zsq_ref[...], axis=0) / n - mu * mu
    rstd = jax.lax.rsqrt(var + 1e-5)
    zn = (z1_ref[...] - mu) * (rstd * g_ref[...]) + bt_ref[...]
    a = jnp.maximum(zn, 0.0)
    tdot = lambda x, w: jax.lax.dot_general(
        x, w, (((1,), (1,)), ((), ())),
        preferred_element_type=jnp.float32)
    mlp = tdot(a, wm2_ref[...]) + bm2_ref[...]
    cap = tdot(sel_ref[...].astype(jnp.bfloat16), wlin_ref[...])
    out_ref[...] = mlp + cap + blin_ref[...]


def _stage1(features, text, W_g1, b_g1, W_g2, W_m1, b_m1):
    trow = text.reshape(NSTEP1, 1, FL)
    row = lambda v: v.reshape(1, -1)
    same, tie, le, diag, dgd, qb, qs, icol = _consts()
    cst = lambda shape: pl.BlockSpec(shape, lambda i: (0,) * len(shape))

    z1, sel, zsum, zsq = pl.pallas_call(
        _k1,
        grid=(NSTEP1,),
        in_specs=[
            pl.BlockSpec((BB, L, D_IN), lambda i: (i, 0, 0)),
            pl.BlockSpec((1, 1, FL), lambda i: (i, 0, 0)),
            cst((FL, 1)),
            cst((FL, FL)),
            cst((FL, FL)),
            cst((FL, FL)),
            cst((FL, FL)),
            cst((D_IN, D_IN)),
            cst((RK, FL)),
            cst((RK, FL)),
            cst((D_IN, D_IN)),
            cst((1, D_IN)),
            cst((1, D_IN)),
            cst((D_IN, D_IN)),
            cst((1, D_IN)),
        ],
        out_specs=[
            pl.BlockSpec((K, BB, D_IN), lambda i: (0, i, 0)),
            pl.BlockSpec((K, BB, D_IN), lambda i: (0, i, 0)),
            pl.BlockSpec((1, 1, D_IN), lambda i: (i, 0, 0)),
            pl.BlockSpec((1, 1, D_IN), lambda i: (i, 0, 0)),
        ],
        out_shape=[
            jax.ShapeDtypeStruct((K, B, D_IN), jnp.float32),
            jax.ShapeDtypeStruct((K, B, D_IN), jnp.float32),
            jax.ShapeDtypeStruct((NSTEP1, 1, D_IN), jnp.float32),
            jax.ShapeDtypeStruct((NSTEP1, 1, D_IN), jnp.float32),
        ],
        compiler_params=pltpu.CompilerParams(
            dimension_semantics=("parallel",)),
    )(features, trow, icol, same, tie, le, diag, dgd, qb, qs,
      W_g1, row(b_g1), W_g2, W_m1, row(b_m1))
    return z1, sel, zsum, zsq


def kernel(features, text, atten, W_g1, b_g1, W_g2, b_g2, W_lin, b_lin,
           W_m1, b_m1, bn_gamma, bn_beta, W_m2, b_m2):
    del atten, b_g2
    z1, sel, zsum, zsq = _stage1(features, text, W_g1, b_g1, W_g2,
                                 W_m1, b_m1)
    z1 = z1.reshape(ROWS, D_IN)
    sel = sel.reshape(ROWS, D_IN)
    row = lambda v: v.reshape(1, -1)
    cst = lambda shape: pl.BlockSpec(shape, lambda i: (0,) * len(shape))

    out = pl.pallas_call(
        _k2,
        grid=(NSTEP2,),
        in_specs=[
            pl.BlockSpec((R2, D_IN), lambda i: (i, 0)),
            pl.BlockSpec((R2, D_IN), lambda i: (i, 0)),
            cst((NSTEP1, 1, D_IN)),
            cst((NSTEP1, 1, D_IN)),
            cst((D_EMB, D_IN)),
            cst((1, D_EMB)),
            cst((D_EMB, D_IN)),
            cst((1, D_EMB)),
            cst((1, D_IN)),
            cst((1, D_IN)),
        ],
        out_specs=pl.BlockSpec((R2, D_EMB), lambda i: (i, 0)),
        out_shape=jax.ShapeDtypeStruct((ROWS, D_EMB), jnp.float32),
        compiler_params=pltpu.CompilerParams(
            dimension_semantics=("parallel",)),
    )(z1, sel, zsum, zsq, W_lin.astype(jnp.bfloat16), row(b_lin),
      W_m2, row(b_m2), row(bn_gamma), row(bn_beta))

    return out.reshape(K, B, D_EMB).transpose(1, 0, 2)


# speedup vs baseline: 1.7462x; 1.0594x over previous
import jax
import jax.numpy as jnp
import numpy as np
from jax.experimental import pallas as pl
from jax.experimental.pallas import tpu as pltpu


B, L, D_IN, D_EMB = 1024, 64, 512, 1024
K = 18
BB = 8
FL = BB * L
RK = BB * K
NSTEP1 = B // BB
ROWS = B * K
R2 = 1024
NSTEP2 = ROWS // R2
NEG = float("-inf")
HIGHEST = jax.lax.Precision.HIGHEST
IMIN = -2147483648


def _consts():
    i = np.arange(FL)
    same = (i[:, None] // L) == (i[None, :] // L)
    tie = same & (i[None, :] < i[:, None])
    le = same & (i[:, None] <= i[None, :])
    diag = i[:, None] == i[None, :]
    q = np.arange(RK)
    qb = (q[:, None] % BB) == (i[None, :] // L)
    qs = np.tile((q[:, None] // BB), (1, FL))
    dg = np.arange(D_IN)
    diag_d = dg[:, None] == dg[None, :]
    f32 = lambda a: jnp.asarray(a, jnp.float32)
    return (f32(same), f32(tie), f32(le), f32(diag), f32(diag_d), f32(qb),
            f32(qs), jnp.asarray(i[:, None], jnp.int32))


def _k1(feat_ref, tr_ref, ic_ref, same_ref, tie_ref, le_ref,
        diag_ref, dgd_ref, qb_ref, qs_ref, wg1_ref, bg1_ref, wg2_ref,
        wm1_ref, bm1_ref, z1_ref, sel_ref, zsum_ref, zsq_ref,
        acc_ref):
    f2 = feat_ref[...].reshape(FL, D_IN)
    h = jnp.maximum(jax.lax.dot_general(
        f2, wg1_ref[...], (((1,), (1,)), ((), ())),
        preferred_element_type=jnp.float32) + bg1_ref[...], 0.0)
    sameb = same_ref[...] != 0.0
    diagb = diag_ref[...] != 0.0
    g2c = jnp.sum(jnp.where(dgd_ref[...] != 0.0, wg2_ref[...], 0.0),
                  axis=1, keepdims=True)
    g2m = jnp.where(
        jax.lax.broadcasted_iota(jnp.int32, (D_IN, 128), 1) == 0, g2c,
        0.0)
    wcol = jnp.dot(h, g2m,
                   preferred_element_type=jnp.float32)[:, 0:1]
    tj = tr_ref[0]
    lr = jax.lax.broadcasted_iota(jnp.int32, (1, FL), 1)
    key = tj * L + (L - 1) - (lr & (L - 1))
    x = key
    for d in (1, 2, 4, 8, 16, 32):
        x = jnp.maximum(x, jnp.where((lr & d) == 0,
                                     pltpu.roll(x, FL - d, 1),
                                     pltpu.roll(x, d, 1)))
    killr = (key == x) | ((lr & (L - 1)) == 0) | (tj == 0)
    killc = jnp.sum(jnp.where(diagb & killr, 1.0, 0.0), axis=1,
                    keepdims=True) > 0.0
    wcol = jnp.where(killc, NEG, wcol)

    wrow = jnp.max(jnp.where(diagb, wcol, NEG), axis=0, keepdims=True)

    beats = (jnp.where(wrow > wcol, same_ref[...], 0.0)
             + jnp.where(wrow == wcol, tie_ref[...], 0.0))
    rank = jnp.sum(beats, axis=1, keepdims=True)
    mcol = rank < float(K)
    cnt = jnp.sum(jnp.where(mcol, le_ref[...], 0.0), axis=0,
                  keepdims=True)
    pos = cnt - 1.0
    prev = jnp.where(
        (jax.lax.broadcasted_iota(jnp.int32, (1, FL), 1) & (L - 1)) == 0,
        0.0, jnp.roll(cnt, 1, axis=1))
    mrow = cnt - prev

    p = jnp.where((pos == qs_ref[...]) & (mrow != 0.0), qb_ref[...], 0.0)
    sel = jnp.dot(p, f2,
                  preferred_element_type=jnp.float32)

    nrm = jnp.sqrt(jnp.sum(sel * sel, axis=1, keepdims=True)) + 1e-8
    seln = sel / nrm
    sel_ref[...] = seln.astype(jnp.bfloat16).reshape(K, BB, D_IN)

    z1 = jax.lax.dot_general(
        seln, wm1_ref[...], (((1,), (1,)), ((), ())),
        preferred_element_type=jnp.float32) + bm1_ref[...]
    z1_ref[...] = z1.reshape(K, BB, D_IN)

    @pl.when(pl.program_id(0) == 0)
    def _():
        acc_ref[...] = jnp.zeros_like(acc_ref)

    acc_ref[0:1] += jnp.sum(z1, axis=0, keepdims=True)
    acc_ref[1:2] += jnp.sum(z1 * z1, axis=0, keepdims=True)

    @pl.when(pl.program_id(0) == NSTEP1 - 1)
    def _():
        zsum_ref[...] = acc_ref[0:1]
        zsq_ref[...] = acc_ref[1:2]


def _k2(z1_ref, sel_ref, zsum_ref, zsq_ref, wlin_ref, blin_ref, wm2_ref,
        bm2_ref, g_ref, bt_ref, out_ref):
    n = float(ROWS)
    mu = zsum_ref[...] / n
    var = zsq_ref[...] / n - mu * mu
    rstd = jax.lax.rsqrt(var + 1e-5)
    zn = (z1_ref[...] - mu) * (rstd * g_ref[...]) + bt_ref[...]
    a = jnp.maximum(zn, 0.0)
    tdot = lambda x, w: jax.lax.dot_general(
        x, w, (((1,), (1,)), ((), ())),
        preferred_element_type=jnp.float32)
    mlp = tdot(a, wm2_ref[...]) + bm2_ref[...]
    cap = tdot(sel_ref[...], wlin_ref[...])
    out_ref[...] = mlp + cap + blin_ref[...]


def _stage1(features, text, W_g1, b_g1, W_g2, W_m1, b_m1):
    trow = text.reshape(NSTEP1, 1, FL)
    row = lambda v: v.reshape(1, -1)
    same, tie, le, diag, dgd, qb, qs, icol = _consts()
    cst = lambda shape: pl.BlockSpec(shape, lambda i: (0,) * len(shape))

    z1, sel, zsum, zsq = pl.pallas_call(
        _k1,
        grid=(NSTEP1,),
        in_specs=[
            pl.BlockSpec((BB, L, D_IN), lambda i: (i, 0, 0)),
            pl.BlockSpec((1, 1, FL), lambda i: (i, 0, 0)),
            cst((FL, 1)),
            cst((FL, FL)),
            cst((FL, FL)),
            cst((FL, FL)),
            cst((FL, FL)),
            cst((D_IN, D_IN)),
            cst((RK, FL)),
            cst((RK, FL)),
            cst((D_IN, D_IN)),
            cst((1, D_IN)),
            cst((1, D_IN)),
            cst((D_IN, D_IN)),
            cst((1, D_IN)),
        ],
        out_specs=[
            pl.BlockSpec((K, BB, D_IN), lambda i: (0, i, 0)),
            pl.BlockSpec((K, BB, D_IN), lambda i: (0, i, 0)),
            pl.BlockSpec((1, D_IN), lambda i: (0, 0)),
            pl.BlockSpec((1, D_IN), lambda i: (0, 0)),
        ],
        out_shape=[
            jax.ShapeDtypeStruct((K, B, D_IN), jnp.float32),
            jax.ShapeDtypeStruct((K, B, D_IN), jnp.bfloat16),
            jax.ShapeDtypeStruct((1, D_IN), jnp.float32),
            jax.ShapeDtypeStruct((1, D_IN), jnp.float32),
        ],
        scratch_shapes=[pltpu.VMEM((2, D_IN), jnp.float32)],
        compiler_params=pltpu.CompilerParams(
            dimension_semantics=("arbitrary",)),
    )(features, trow, icol, same, tie, le, diag, dgd, qb, qs,
      W_g1, row(b_g1), W_g2, W_m1, row(b_m1))
    return z1, sel, zsum, zsq


def kernel(features, text, atten, W_g1, b_g1, W_g2, b_g2, W_lin, b_lin,
           W_m1, b_m1, bn_gamma, bn_beta, W_m2, b_m2):
    del atten, b_g2
    z1, sel, zsum, zsq = _stage1(features, text, W_g1, b_g1, W_g2,
                                 W_m1, b_m1)
    z1 = z1.reshape(ROWS, D_IN)
    sel = sel.reshape(ROWS, D_IN)
    row = lambda v: v.reshape(1, -1)
    cst = lambda shape: pl.BlockSpec(shape, lambda i: (0,) * len(shape))

    out = pl.pallas_call(
        _k2,
        grid=(NSTEP2,),
        in_specs=[
            pl.BlockSpec((R2, D_IN), lambda i: (i, 0)),
            pl.BlockSpec((R2, D_IN), lambda i: (i, 0)),
            cst((1, D_IN)),
            cst((1, D_IN)),
            cst((D_EMB, D_IN)),
            cst((1, D_EMB)),
            cst((D_EMB, D_IN)),
            cst((1, D_EMB)),
            cst((1, D_IN)),
            cst((1, D_IN)),
        ],
        out_specs=pl.BlockSpec((R2, D_EMB), lambda i: (i, 0)),
        out_shape=jax.ShapeDtypeStruct((ROWS, D_EMB), jnp.float32),
        compiler_params=pltpu.CompilerParams(
            dimension_semantics=("parallel",)),
    )(z1, sel, zsum, zsq, W_lin.astype(jnp.bfloat16), row(b_lin),
      W_m2, row(b_m2), row(bn_gamma), row(bn_beta))

    return out.reshape(K, B, D_EMB).transpose(1, 0, 2)
